# baseline (speedup 1.0000x reference)
import functools

import jax
import jax.numpy as jnp
from jax import lax
from jax.experimental import pallas as pl
from jax.experimental.pallas import tpu as pltpu
from jax.experimental.pallas import tpu_sc as plsc

_H = 1024
_Q = 4
_NP = 2 ** _Q
_G = 64


def _make_sc_kernel(n_tokens):
    info = plsc.get_sparse_core_info()
    nw = info.num_cores * info.num_subcores
    tpw = n_tokens // nw
    ng = tpw // _G
    qpg = _G // _Q
    mesh = plsc.VectorSubcoreMesh(core_axis_name="c", subcore_axis_name="s")

    @functools.partial(
        pl.kernel,
        mesh=mesh,
        out_type=jax.ShapeDtypeStruct((n_tokens * _H,), jnp.float32),
        scratch_types=[
            pltpu.VMEM_SHARED((info.num_subcores, tpw), jnp.int32),
            pltpu.VMEM_SHARED((_NP, _Q * _H), jnp.float32),
            pltpu.SMEM((tpw,), jnp.int32),
            pltpu.VMEM((2 * _H,), jnp.float32),
            pltpu.VMEM((_Q * _H,), jnp.float32),
            pltpu.SemaphoreType.DMA,
        ],
    )
    def k(seg_hbm, table_hbm, out_hbm, idx_sh, pat_sh, seg_s, table_v,
          pat_v, sem):
        sid = lax.axis_index("s")
        wid = sid * info.num_cores + lax.axis_index("c")
        base = wid * tpw
        pltpu.sync_copy(seg_hbm.at[pl.ds(base, tpw)], idx_sh.at[sid])
        pltpu.sync_copy(idx_sh.at[sid], seg_s)
        pltpu.sync_copy(table_hbm, table_v)

        for j in range(_Q):
            bit = (sid >> (_Q - 1 - j)) & 1
            src_base = bit * _H

            def cp(c, carry, _dst_base=j * _H, _src_base=src_base):
                pat_v[pl.ds(_dst_base + 16 * c, 16)] = (
                    table_v[pl.ds(_src_base + 16 * c, 16)])
                return carry

            lax.fori_loop(0, _H // 16, cp, 0)
        pltpu.sync_copy(pat_v, pat_sh.at[sid])
        plsc.subcore_barrier()

        def drain_one_group():
            pltpu.make_async_copy(
                out_hbm.at[pl.ds(base * _H, _G * _H)],
                out_hbm.at[pl.ds(base * _H, _G * _H)],
                sem).wait()

        def quad(i, c):
            p = (seg_s[_Q * i] * 8 + seg_s[_Q * i + 1] * 4
                 + seg_s[_Q * i + 2] * 2 + seg_s[_Q * i + 3])
            pltpu.make_async_copy(
                pat_sh.at[p],
                out_hbm.at[pl.ds((base + _Q * i) * _H, _Q * _H)],
                sem).start()
            return c

        def grp(g, c):
            lax.fori_loop(g * qpg, (g + 1) * qpg, quad, 0)

            @pl.when(g > 0)
            def _():
                drain_one_group()

            return c

        lax.fori_loop(0, ng, grp, 0)
        drain_one_group()

    return k


def kernel(segments, table):
    b, s = segments.shape
    n = b * s
    out = _make_sc_kernel(n)(segments.reshape(n), table.reshape(2 * _H))
    return out.reshape(b, s, _H)


# speedup vs baseline: 1.1345x; 1.1345x over previous
import functools

import jax
import jax.numpy as jnp
from jax import lax
from jax.experimental import pallas as pl
from jax.experimental.pallas import tpu as pltpu
from jax.experimental.pallas import tpu_sc as plsc

_H = 1024
_Q = 2
_NP = 2 ** _Q
_G = 64


def _make_sc_kernel(n_tokens):
    info = plsc.get_sparse_core_info()
    nw = info.num_cores * info.num_subcores
    tpw = n_tokens // nw
    ng = tpw // _G
    qpg = _G // _Q
    mesh = plsc.VectorSubcoreMesh(core_axis_name="c", subcore_axis_name="s")

    @functools.partial(
        pl.kernel,
        mesh=mesh,
        out_type=jax.ShapeDtypeStruct((n_tokens * _H,), jnp.float32),
        scratch_types=[
            pltpu.VMEM_SHARED((info.num_subcores, tpw), jnp.int32),
            pltpu.SMEM((tpw,), jnp.int32),
            pltpu.VMEM((2 * _H,), jnp.float32),
            pltpu.VMEM((_NP, _Q * _H), jnp.float32),
            pltpu.SemaphoreType.DMA,
        ],
    )
    def k(seg_hbm, table_hbm, out_hbm, idx_sh, seg_s, table_v, pat_v, sem):
        sid = lax.axis_index("s")
        wid = sid * info.num_cores + lax.axis_index("c")
        base = wid * tpw
        pltpu.sync_copy(seg_hbm.at[pl.ds(base, tpw)], idx_sh.at[sid])
        pltpu.sync_copy(idx_sh.at[sid], seg_s)
        pltpu.sync_copy(table_hbm, table_v)

        for p in range(_NP):
            for j in range(_Q):
                bit = (p >> (_Q - 1 - j)) & 1

                def cp(c, carry, _p=p, _dst=j * _H, _src=bit * _H):
                    pat_v[_p, pl.ds(_dst + 16 * c, 16)] = (
                        table_v[pl.ds(_src + 16 * c, 16)])
                    return carry

                lax.fori_loop(0, _H // 16, cp, 0)

        def drain_one_group():
            pltpu.make_async_copy(
                out_hbm.at[pl.ds(base * _H, _G * _H)],
                out_hbm.at[pl.ds(base * _H, _G * _H)],
                sem).wait()

        def quad(i, c):
            p = seg_s[_Q * i] * 2 + seg_s[_Q * i + 1]
            pltpu.make_async_copy(
                pat_v.at[p],
                out_hbm.at[pl.ds((base + _Q * i) * _H, _Q * _H)],
                sem).start()
            return c

        def grp(g, c):
            lax.fori_loop(g * qpg, (g + 1) * qpg, quad, 0)

            @pl.when(g > 0)
            def _():
                drain_one_group()

            return c

        lax.fori_loop(0, ng, grp, 0)
        drain_one_group()

    return k


def kernel(segments, table):
    b, s = segments.shape
    n = b * s
    out = _make_sc_kernel(n)(segments.reshape(n), table.reshape(2 * _H))
    return out.reshape(b, s, _H)
